# baseline (speedup 1.0000x reference)
import functools

import jax
import jax.numpy as jnp
from jax import lax
from jax.experimental import pallas as pl
from jax.experimental.pallas import tpu as pltpu
from jax.experimental.pallas import tpu_sc as plsc

def _oddeven_merge_sort_pairs(n):
    pairs = []
    p = 1
    while p < n:
        k = p
        while k >= 1:
            for j in range(k % p, n - k, 2 * k):
                for i in range(0, min(k, n - j - k)):
                    if (i + j) // (p * 2) == (i + j + k) // (p * 2):
                        pairs.append((i + j, i + j + k))
            k //= 2
        p *= 2
    return pairs


def _pruned_sort_ops(n_real, n_pad):
    pos = list(range(n_real)) + [None] * (n_pad - n_real)
    ops = []
    for (i, j) in _oddeven_merge_sort_pairs(n_pad):
        a, b = pos[i], pos[j]
        if a is not None and b is not None:
            ops.append((a, b))
        elif a is None and b is not None:
            pos[i], pos[j] = b, None
    return ops, [p for p in pos if p is not None]


def _prune_dead(ops, needed):
    needed = set(needed)
    kept = []
    for (a, b) in reversed(ops):
        if a in needed or b in needed:
            kept.append((a, b))
            needed.add(a)
            needed.add(b)
    return kept[::-1]


_OPS16, _ORDER16 = _pruned_sort_ops(16, 16)
_OPS16 = _prune_dead(_OPS16, _ORDER16[:10])
_OPS10, _ORDER10 = _pruned_sort_ops(10, 16)

N_USER = 1024
M_ITEM = 100000
DIM = 32
BLK = 2048
NBLK = 49
M_PAD = NBLK * BLK
LAST_VALID = M_ITEM - (NBLK - 1) * BLK
K = 10
LANES = 128
NGRP = BLK // LANES
PCAP = 512
PADKEY = 131071 << 10

_SC_NC = 2
_SC_NS = 16
_SC_NW = _SC_NC * _SC_NS
_ROWS_PER_W = N_USER // _SC_NW


def _sc_gather_pos(items_emb, pos_items):
    mesh = plsc.VectorSubcoreMesh(core_axis_name="c", subcore_axis_name="s")

    @functools.partial(
        pl.kernel,
        mesh=mesh,
        out_type=jax.ShapeDtypeStruct((N_USER, DIM), jnp.float32),
        compiler_params=pltpu.CompilerParams(use_tc_tiling_on_sc=False),
        scratch_types=[
            pltpu.VMEM((_ROWS_PER_W,), jnp.int32),
            pltpu.VMEM((_ROWS_PER_W, DIM), jnp.float32),
            pltpu.SemaphoreType.DMA,
        ],
    )
    def gather_k(table_hbm, idx_hbm, out_hbm, idx_v, rows_v, sem):
        wid = lax.axis_index("s") * _SC_NC + lax.axis_index("c")
        base = pl.multiple_of(wid * _ROWS_PER_W, _ROWS_PER_W)
        pltpu.sync_copy(idx_hbm.at[pl.ds(base, _ROWS_PER_W)], idx_v)
        pltpu.async_copy(table_hbm.at[idx_v], rows_v, sem).wait()
        pltpu.sync_copy(rows_v, out_hbm.at[pl.ds(base, _ROWS_PER_W)])

    return gather_k(items_emb, pos_items.astype(jnp.int32))


def _tc_body(pairs_ref, offs_ref, offs_ov_ref, users_ref, items_ref,
             pairs_blk_ref, pos_ref, out_ref, s_ref, t_ref, k_ref):
    pid = pl.program_id(0)

    @pl.when(pid == 0)
    def _init():
        t_ref[...] = jnp.full((K, N_USER, LANES), -jnp.inf, jnp.float32)

    w = pairs_blk_ref[0]
    u_flat = w & 1023
    col_flat = (w >> 10) - pid * BLK
    a2 = jnp.where(
        lax.broadcasted_iota(jnp.int32, (N_USER, PCAP), 0) == u_flat,
        jnp.float32(1.0), jnp.float32(0.0)).astype(jnp.bfloat16)
    b2 = jnp.where(
        lax.broadcasted_iota(jnp.int32, (BLK, PCAP), 0) == col_flat,
        jnp.float32(-2048.0), jnp.float32(0.0)).astype(jnp.bfloat16)
    s_ref[...] = (
        lax.dot_general(
            users_ref[...].astype(jnp.bfloat16), items_ref[...],
            (((1,), (1,)), ((), ())),
            preferred_element_type=jnp.float32)
        + lax.dot_general(
            a2, b2, (((1,), (1,)), ((), ())),
            preferred_element_type=jnp.float32))

    @pl.when(pid == NBLK - 1)
    def _mask_pad():
        s_ref[:, LAST_VALID:] = jnp.full(
            (N_USER, BLK - LAST_VALID), -jnp.inf, jnp.float32)

    start = offs_ov_ref[pid]
    end = offs_ref[pid + 1]

    sub_iota = lax.broadcasted_iota(jnp.int32, (8, 128), 0)
    lane_iota = lax.broadcasted_iota(jnp.int32, (8, 128), 1)
    flat_iota = sub_iota * 128 + lane_iota

    def pair_body(j, carry):
        v = pairs_ref[j]
        u = v & 1023
        col = (v >> 10) - pid * BLK
        ua = pl.multiple_of(u & ~7, 8)
        ca = pl.multiple_of(col & ~127, 128)
        patch = s_ref[pl.ds(ua, 8), pl.ds(ca, 128)]
        hit = flat_iota == ((u & 7) * 128 + (col & 127))
        s_ref[pl.ds(ua, 8), pl.ds(ca, 128)] = jnp.where(
            hit, patch - jnp.float32(2048.0), patch)
        return carry

    lax.fori_loop(start, end, pair_body, 0)

    vs = [s_ref[:, g * LANES:(g + 1) * LANES] for g in range(NGRP)]
    for (a, b) in _OPS16:
        hi = jnp.maximum(vs[a], vs[b])
        lo = jnp.minimum(vs[a], vs[b])
        vs[a], vs[b] = hi, lo
    blk_top = [vs[_ORDER16[i]] for i in range(K)]
    cur = [t_ref[d] for d in range(K)]
    c = [jnp.maximum(cur[i], blk_top[K - 1 - i]) for i in range(K)]
    for (a, b) in _OPS10:
        hi = jnp.maximum(c[a], c[b])
        lo = jnp.minimum(c[a], c[b])
        c[a], c[b] = hi, lo
    for d in range(K):
        t_ref[d] = c[_ORDER10[d]]

    @pl.when(pid == NBLK - 1)
    def _finish():
        for d in range(K):
            b = lax.bitcast_convert_type(t_ref[d], jnp.int32)
            k_ref[d] = b ^ (lax.shift_right_arithmetic(b, 31) & 0x7FFFFFFF)

        sign = jnp.int32(-2147483648)

        def bit_body(i, r):
            bit = 31 - i
            c = r | (jnp.int32(1) << bit)
            cs = (c ^ sign)[:, None]
            cnt = jnp.zeros((N_USER,), jnp.int32)
            for d in range(K):
                cnt = cnt + jnp.sum(
                    (k_ref[d] >= cs).astype(jnp.int32), axis=1)
            return jnp.where(cnt >= K, c, r)

        r = lax.fori_loop(0, 32, bit_body,
                          jnp.full((N_USER,), jnp.int32(0)))
        sk = r ^ sign
        nb = sk ^ (lax.shift_right_arithmetic(sk, 31) & 0x7FFFFFFF)
        neg = lax.bitcast_convert_type(nb, jnp.float32)
        pos = jnp.sum(users_ref[...] * pos_ref[...], axis=1)
        loss = neg - pos
        loss = jnp.where(loss < 0, jnp.exp(loss) - 1.0, loss)
        out_ref[...] = jnp.sum(loss)[None, None]


def _tc_grid_spec():
    return pltpu.PrefetchScalarGridSpec(
        num_scalar_prefetch=3,
        grid=(NBLK,),
        in_specs=[
            pl.BlockSpec((N_USER, DIM), lambda i, *_: (0, 0)),
            pl.BlockSpec((BLK, DIM), lambda i, *_: (i, 0)),
            pl.BlockSpec((1, 1, PCAP), lambda i, *_: (i, 0, 0)),
            pl.BlockSpec((N_USER, DIM), lambda i, *_: (0, 0)),
        ],
        out_specs=pl.BlockSpec((1, 1), lambda i, *_: (0, 0)),
        scratch_shapes=[
            pltpu.VMEM((N_USER, BLK), jnp.float32),
            pltpu.VMEM((K, N_USER, LANES), jnp.float32),
            pltpu.VMEM((K, N_USER, LANES), jnp.int32),
        ],
    )


def kernel(users_emb, items_emb, ignore_users, ignore_items, pos_items):
    pos_emb = _sc_gather_pos(items_emb, pos_items)

    n_pairs = ignore_users.shape[0]
    keys = ignore_items.astype(jnp.int32) * 1024 + ignore_users.astype(jnp.int32)
    keys = jnp.sort(keys)
    bounds = (jnp.arange(NBLK + 1, dtype=jnp.int32) * (BLK * 1024))
    offs = jnp.searchsorted(keys, bounds).astype(jnp.int32)
    counts = offs[1:] - offs[:-1]
    offs_ov = (offs[:-1] + jnp.minimum(counts, PCAP)).astype(jnp.int32)
    pidx = jnp.arange(n_pairs, dtype=jnp.int32)
    blk_of = (jnp.searchsorted(offs, pidx, side="right") - 1).astype(jnp.int32)
    rank = pidx - offs[blk_of]
    dest = jnp.where(rank < PCAP, blk_of * PCAP + rank, NBLK * PCAP)
    pairs_blk = jnp.full((NBLK * PCAP,), PADKEY, jnp.int32)
    pairs_blk = pairs_blk.at[dest].set(keys, mode="drop")
    pairs_blk = pairs_blk.reshape(NBLK, 1, PCAP)

    items_pad = jnp.concatenate(
        [items_emb.astype(jnp.bfloat16),
         jnp.zeros((M_PAD - M_ITEM, DIM), jnp.bfloat16)], axis=0)

    out = pl.pallas_call(
        _tc_body,
        grid_spec=_tc_grid_spec(),
        out_shape=jax.ShapeDtypeStruct((1, 1), jnp.float32),
    )(keys, offs, offs_ov, users_emb, items_pad, pairs_blk, pos_emb)
    return out[0, 0]


# speedup vs baseline: 1.5889x; 1.5889x over previous
import functools

import jax
import jax.numpy as jnp
from jax import lax
from jax.experimental import pallas as pl
from jax.experimental.pallas import tpu as pltpu
from jax.experimental.pallas import tpu_sc as plsc

def _oddeven_merge_sort_pairs(n):
    pairs = []
    p = 1
    while p < n:
        k = p
        while k >= 1:
            for j in range(k % p, n - k, 2 * k):
                for i in range(0, min(k, n - j - k)):
                    if (i + j) // (p * 2) == (i + j + k) // (p * 2):
                        pairs.append((i + j, i + j + k))
            k //= 2
        p *= 2
    return pairs


def _pruned_sort_ops(n_real, n_pad):
    pos = list(range(n_real)) + [None] * (n_pad - n_real)
    ops = []
    for (i, j) in _oddeven_merge_sort_pairs(n_pad):
        a, b = pos[i], pos[j]
        if a is not None and b is not None:
            ops.append((a, b))
        elif a is None and b is not None:
            pos[i], pos[j] = b, None
    return ops, [p for p in pos if p is not None]


def _prune_dead(ops, needed):
    needed = set(needed)
    kept = []
    for (a, b) in reversed(ops):
        if a in needed or b in needed:
            kept.append((a, b))
            needed.add(a)
            needed.add(b)
    return kept[::-1]


_OPS16, _ORDER16 = _pruned_sort_ops(16, 16)
_OPS16 = _prune_dead(_OPS16, _ORDER16[:10])
_OPS10, _ORDER10 = _pruned_sort_ops(10, 16)

N_USER = 1024
M_ITEM = 100000
DIM = 32
BLK = 2048
NBLK = 49
M_PAD = NBLK * BLK
LAST_VALID = M_ITEM - (NBLK - 1) * BLK
K = 10
LANES = 128
NGRP = BLK // LANES

_SC_NC = 2
_SC_NS = 16
_SC_NW = _SC_NC * _SC_NS
_ROWS_PER_W = N_USER // _SC_NW


def _sc_gather_pos(items_emb, pos_items):
    mesh = plsc.VectorSubcoreMesh(core_axis_name="c", subcore_axis_name="s")

    @functools.partial(
        pl.kernel,
        mesh=mesh,
        out_type=jax.ShapeDtypeStruct((N_USER, DIM), jnp.float32),
        compiler_params=pltpu.CompilerParams(use_tc_tiling_on_sc=False),
        scratch_types=[
            pltpu.VMEM((_ROWS_PER_W,), jnp.int32),
            pltpu.VMEM((_ROWS_PER_W, DIM), jnp.float32),
            pltpu.SemaphoreType.DMA,
        ],
    )
    def gather_k(table_hbm, idx_hbm, out_hbm, idx_v, rows_v, sem):
        wid = lax.axis_index("s") * _SC_NC + lax.axis_index("c")
        base = pl.multiple_of(wid * _ROWS_PER_W, _ROWS_PER_W)
        pltpu.sync_copy(idx_hbm.at[pl.ds(base, _ROWS_PER_W)], idx_v)
        pltpu.async_copy(table_hbm.at[idx_v], rows_v, sem).wait()
        pltpu.sync_copy(rows_v, out_hbm.at[pl.ds(base, _ROWS_PER_W)])

    return gather_k(items_emb, pos_items.astype(jnp.int32))


def _tc_body(pairs_ref, offs_ref, users_ref, items_ref,
             pos_ref, out_ref, s_ref, t_ref, k_ref):
    pid = pl.program_id(0)

    @pl.when(pid == 0)
    def _init():
        t_ref[...] = jnp.full((K, N_USER, LANES), -jnp.inf, jnp.float32)

    s_ref[...] = lax.dot_general(
        users_ref[...].astype(jnp.bfloat16), items_ref[...],
        (((1,), (1,)), ((), ())),
        preferred_element_type=jnp.float32)

    @pl.when(pid == NBLK - 1)
    def _mask_pad():
        s_ref[:, LAST_VALID:] = jnp.full(
            (N_USER, BLK - LAST_VALID), -jnp.inf, jnp.float32)

    start = offs_ref[pid]
    end = offs_ref[pid + 1]

    sub_iota = lax.broadcasted_iota(jnp.int32, (8, 128), 0)
    lane_iota = lax.broadcasted_iota(jnp.int32, (8, 128), 1)
    flat_iota = sub_iota * 128 + lane_iota

    def pair_body(j, carry):
        v = pairs_ref[j]
        u = v & 1023
        col = (v >> 10) - pid * BLK
        ua = pl.multiple_of(u & ~7, 8)
        ca = pl.multiple_of(col & ~127, 128)
        patch = s_ref[pl.ds(ua, 8), pl.ds(ca, 128)]
        hit = flat_iota == ((u & 7) * 128 + (col & 127))
        s_ref[pl.ds(ua, 8), pl.ds(ca, 128)] = jnp.where(
            hit, jnp.float32(-1024.0), patch)
        return carry

    lax.fori_loop(start, end, pair_body, 0)

    vs = [s_ref[:, g * LANES:(g + 1) * LANES] for g in range(NGRP)]
    for (a, b) in _OPS16:
        hi = jnp.maximum(vs[a], vs[b])
        lo = jnp.minimum(vs[a], vs[b])
        vs[a], vs[b] = hi, lo
    blk_top = [vs[_ORDER16[i]] for i in range(K)]
    cur = [t_ref[d] for d in range(K)]
    c = [jnp.maximum(cur[i], blk_top[K - 1 - i]) for i in range(K)]
    for (a, b) in _OPS10:
        hi = jnp.maximum(c[a], c[b])
        lo = jnp.minimum(c[a], c[b])
        c[a], c[b] = hi, lo
    for d in range(K):
        t_ref[d] = c[_ORDER10[d]]

    @pl.when(pid == NBLK - 1)
    def _finish():
        for d in range(K):
            b = lax.bitcast_convert_type(t_ref[d], jnp.int32)
            k_ref[d] = b ^ (lax.shift_right_arithmetic(b, 31) & 0x7FFFFFFF)

        sign = jnp.int32(-2147483648)

        def bit_body(i, r):
            bit = 31 - i
            c = r | (jnp.int32(1) << bit)
            cs = (c ^ sign)[:, None]
            cnt = jnp.zeros((N_USER,), jnp.int32)
            for d in range(K):
                cnt = cnt + jnp.sum(
                    (k_ref[d] >= cs).astype(jnp.int32), axis=1)
            return jnp.where(cnt >= K, c, r)

        r = lax.fori_loop(0, 32, bit_body,
                          jnp.full((N_USER,), jnp.int32(0)))
        sk = r ^ sign
        nb = sk ^ (lax.shift_right_arithmetic(sk, 31) & 0x7FFFFFFF)
        neg = lax.bitcast_convert_type(nb, jnp.float32)
        pos = jnp.sum(users_ref[...] * pos_ref[...], axis=1)
        loss = neg - pos
        loss = jnp.where(loss < 0, jnp.exp(loss) - 1.0, loss)
        out_ref[...] = jnp.sum(loss)[None, None]


def _tc_grid_spec():
    return pltpu.PrefetchScalarGridSpec(
        num_scalar_prefetch=2,
        grid=(NBLK,),
        in_specs=[
            pl.BlockSpec((N_USER, DIM), lambda i, *_: (0, 0)),
            pl.BlockSpec((BLK, DIM), lambda i, *_: (i, 0)),
            pl.BlockSpec((N_USER, DIM), lambda i, *_: (0, 0)),
        ],
        out_specs=pl.BlockSpec((1, 1), lambda i, *_: (0, 0)),
        scratch_shapes=[
            pltpu.VMEM((N_USER, BLK), jnp.float32),
            pltpu.VMEM((K, N_USER, LANES), jnp.float32),
            pltpu.VMEM((K, N_USER, LANES), jnp.int32),
        ],
    )


def kernel(users_emb, items_emb, ignore_users, ignore_items, pos_items):
    pos_emb = _sc_gather_pos(items_emb, pos_items)

    keys = ignore_items.astype(jnp.int32) * 1024 + ignore_users.astype(jnp.int32)
    keys = jnp.sort(keys)
    bounds = (jnp.arange(NBLK + 1, dtype=jnp.int32) * (BLK * 1024))
    offs = jnp.searchsorted(keys, bounds).astype(jnp.int32)

    items_pad = jnp.concatenate(
        [items_emb.astype(jnp.bfloat16),
         jnp.zeros((M_PAD - M_ITEM, DIM), jnp.bfloat16)], axis=0)

    out = pl.pallas_call(
        _tc_body,
        grid_spec=_tc_grid_spec(),
        out_shape=jax.ShapeDtypeStruct((1, 1), jnp.float32),
    )(keys, offs, users_emb, items_pad, pos_emb)
    return out[0, 0]


# speedup vs baseline: 1.6414x; 1.0330x over previous
import functools

import jax
import jax.numpy as jnp
from jax import lax
from jax.experimental import pallas as pl
from jax.experimental.pallas import tpu as pltpu
from jax.experimental.pallas import tpu_sc as plsc

def _oddeven_merge_sort_pairs(n):
    pairs = []
    p = 1
    while p < n:
        k = p
        while k >= 1:
            for j in range(k % p, n - k, 2 * k):
                for i in range(0, min(k, n - j - k)):
                    if (i + j) // (p * 2) == (i + j + k) // (p * 2):
                        pairs.append((i + j, i + j + k))
            k //= 2
        p *= 2
    return pairs


def _pruned_sort_ops(n_real, n_pad):
    pos = list(range(n_real)) + [None] * (n_pad - n_real)
    ops = []
    for (i, j) in _oddeven_merge_sort_pairs(n_pad):
        a, b = pos[i], pos[j]
        if a is not None and b is not None:
            ops.append((a, b))
        elif a is None and b is not None:
            pos[i], pos[j] = b, None
    return ops, [p for p in pos if p is not None]


def _prune_dead(ops, needed):
    needed = set(needed)
    kept = []
    for (a, b) in reversed(ops):
        if a in needed or b in needed:
            kept.append((a, b))
            needed.add(a)
            needed.add(b)
    return kept[::-1]


_OPS16, _ORDER16 = _pruned_sort_ops(16, 16)
_OPS16 = _prune_dead(_OPS16, _ORDER16[:10])
_OPS10, _ORDER10 = _pruned_sort_ops(10, 16)

N_USER = 1024
M_ITEM = 100000
DIM = 32
BLK = 2048
NBLK = 49
M_PAD = NBLK * BLK
LAST_VALID = M_ITEM - (NBLK - 1) * BLK
K = 10
LANES = 128
NGRP = BLK // LANES

_SC_NC = 2
_SC_NS = 16
_SC_NW = _SC_NC * _SC_NS
_ROWS_PER_W = N_USER // _SC_NW


def _sc_gather_pos(items_emb, pos_items):
    mesh = plsc.VectorSubcoreMesh(core_axis_name="c", subcore_axis_name="s")

    @functools.partial(
        pl.kernel,
        mesh=mesh,
        out_type=jax.ShapeDtypeStruct((N_USER, DIM), jnp.float32),
        compiler_params=pltpu.CompilerParams(use_tc_tiling_on_sc=False),
        scratch_types=[
            pltpu.VMEM((_ROWS_PER_W,), jnp.int32),
            pltpu.VMEM((_ROWS_PER_W, DIM), jnp.float32),
            pltpu.SemaphoreType.DMA,
        ],
    )
    def gather_k(table_hbm, idx_hbm, out_hbm, idx_v, rows_v, sem):
        wid = lax.axis_index("s") * _SC_NC + lax.axis_index("c")
        base = pl.multiple_of(wid * _ROWS_PER_W, _ROWS_PER_W)
        pltpu.sync_copy(idx_hbm.at[pl.ds(base, _ROWS_PER_W)], idx_v)
        pltpu.async_copy(table_hbm.at[idx_v], rows_v, sem).wait()
        pltpu.sync_copy(rows_v, out_hbm.at[pl.ds(base, _ROWS_PER_W)])

    return gather_k(items_emb, pos_items.astype(jnp.int32))


def _tc_body(pairs_ref, offs_ref, users_ref, items_ref,
             pos_ref, out_ref, s_ref, t_ref, k_ref):
    pid = pl.program_id(0)

    @pl.when(pid == 0)
    def _init():
        t_ref[...] = jnp.full((K, N_USER, LANES), -jnp.inf, jnp.float32)

    s_ref[...] = lax.dot_general(
        users_ref[...].astype(jnp.bfloat16), items_ref[...],
        (((1,), (1,)), ((), ())),
        preferred_element_type=jnp.float32)

    @pl.when(pid == NBLK - 1)
    def _mask_pad():
        s_ref[:, LAST_VALID:] = jnp.full(
            (N_USER, BLK - LAST_VALID), -jnp.inf, jnp.float32)

    start = offs_ref[pid]
    end = offs_ref[pid + 1]

    sub_iota = lax.broadcasted_iota(jnp.int32, (8, 128), 0)
    lane_iota = lax.broadcasted_iota(jnp.int32, (8, 128), 1)

    def pair_body(j, carry):
        v = pairs_ref[j]
        u = v & 1023
        col = (v >> 10) - pid * BLK
        ua = pl.multiple_of(u & ~7, 8)
        ca = pl.multiple_of(col & ~127, 128)
        patch = s_ref[pl.ds(ua, 8), pl.ds(ca, 128)]
        hit = (sub_iota == (u & 7)) & (lane_iota == (col & 127))
        s_ref[pl.ds(ua, 8), pl.ds(ca, 128)] = jnp.where(
            hit, jnp.float32(-1024.0), patch)
        return carry

    lax.fori_loop(start, end, pair_body, 0)

    vs = [s_ref[:, g * LANES:(g + 1) * LANES] for g in range(NGRP)]
    for (a, b) in _OPS16:
        hi = jnp.maximum(vs[a], vs[b])
        lo = jnp.minimum(vs[a], vs[b])
        vs[a], vs[b] = hi, lo
    blk_top = [vs[_ORDER16[i]] for i in range(K)]
    cur = [t_ref[d] for d in range(K)]
    c = [jnp.maximum(cur[i], blk_top[K - 1 - i]) for i in range(K)]
    for (a, b) in _OPS10:
        hi = jnp.maximum(c[a], c[b])
        lo = jnp.minimum(c[a], c[b])
        c[a], c[b] = hi, lo
    for d in range(K):
        t_ref[d] = c[_ORDER10[d]]

    @pl.when(pid == NBLK - 1)
    def _finish():
        for d in range(K):
            b = lax.bitcast_convert_type(t_ref[d], jnp.int32)
            k_ref[d] = b ^ (lax.shift_right_arithmetic(b, 31) & 0x7FFFFFFF)

        sign = jnp.int32(-2147483648)

        def bit_body(i, r):
            bit = 31 - i
            c = r | (jnp.int32(1) << bit)
            cs = (c ^ sign)[:, None]
            cnt = jnp.zeros((N_USER,), jnp.int32)
            for d in range(K):
                cnt = cnt + jnp.sum(
                    (k_ref[d] >= cs).astype(jnp.int32), axis=1)
            return jnp.where(cnt >= K, c, r)

        r = lax.fori_loop(0, 32, bit_body,
                          jnp.full((N_USER,), jnp.int32(0)))
        sk = r ^ sign
        nb = sk ^ (lax.shift_right_arithmetic(sk, 31) & 0x7FFFFFFF)
        neg = lax.bitcast_convert_type(nb, jnp.float32)
        pos = jnp.sum(users_ref[...] * pos_ref[...], axis=1)
        loss = neg - pos
        loss = jnp.where(loss < 0, jnp.exp(loss) - 1.0, loss)
        out_ref[...] = jnp.sum(loss)[None, None]


def _tc_grid_spec():
    return pltpu.PrefetchScalarGridSpec(
        num_scalar_prefetch=2,
        grid=(NBLK,),
        in_specs=[
            pl.BlockSpec((N_USER, DIM), lambda i, *_: (0, 0)),
            pl.BlockSpec((BLK, DIM), lambda i, *_: (i, 0)),
            pl.BlockSpec((N_USER, DIM), lambda i, *_: (0, 0)),
        ],
        out_specs=pl.BlockSpec((1, 1), lambda i, *_: (0, 0)),
        scratch_shapes=[
            pltpu.VMEM((N_USER, BLK), jnp.float32),
            pltpu.VMEM((K, N_USER, LANES), jnp.float32),
            pltpu.VMEM((K, N_USER, LANES), jnp.int32),
        ],
    )


def kernel(users_emb, items_emb, ignore_users, ignore_items, pos_items):
    pos_emb = _sc_gather_pos(items_emb, pos_items)

    keys = ignore_items.astype(jnp.int32) * 1024 + ignore_users.astype(jnp.int32)
    keys = jnp.sort(keys)
    bounds = (jnp.arange(NBLK + 1, dtype=jnp.int32) * (BLK * 1024))
    offs = jnp.searchsorted(keys, bounds).astype(jnp.int32)

    items_pad = jnp.concatenate(
        [items_emb.astype(jnp.bfloat16),
         jnp.zeros((M_PAD - M_ITEM, DIM), jnp.bfloat16)], axis=0)

    out = pl.pallas_call(
        _tc_body,
        grid_spec=_tc_grid_spec(),
        out_shape=jax.ShapeDtypeStruct((1, 1), jnp.float32),
    )(keys, offs, users_emb, items_pad, pos_emb)
    return out[0, 0]
